# baseline (speedup 1.0000x reference)
import functools

import jax
import jax.numpy as jnp
from jax import lax
from jax.experimental import pallas as pl
from jax.experimental.pallas import tpu as pltpu
from jax.experimental.pallas import tpu_sc as plsc

N = 10000
D = 128
E = 320000

NC = 2
NS = 16
NW = NC * NS

EC = 128
CH = 80
IB = 8
C0 = 40
C1 = 120
TCH = NS * (C0 + C1)
GP = EC // 16
WB = 128
EPT = CH * EC
E_PAD = NW * EPT
NPAD = 10240
RPS = NPAD // NS

_mesh = plsc.VectorSubcoreMesh(core_axis_name="c", subcore_axis_name="s")
_sc_params = pltpu.CompilerParams(needs_layout_passes=False,
                                  use_tc_tiling_on_sc=False)


@functools.partial(
    pl.kernel,
    out_type=jax.ShapeDtypeStruct((NC, NPAD), jnp.float32),
    mesh=_mesh,
    compiler_params=_sc_params,
    scratch_types=[
        pltpu.VMEM((CH, EC), jnp.int32),
        pltpu.VMEM((NPAD,), jnp.float32),
        pltpu.VMEM((RPS,), jnp.float32),
        pltpu.VMEM((RPS,), jnp.float32),
        pltpu.VMEM_SHARED((NS, NPAD), jnp.float32),
    ],
)
def _deg_kernel(col_hbm, deg_out, cbuf, hist, tbuf, acc, sh):
    c = lax.axis_index("c")
    s = lax.axis_index("s")
    wid = c * NS + s
    pltpu.sync_copy(col_hbm.at[wid], cbuf)
    zeros = jnp.zeros((16,), jnp.float32)
    ones = jnp.ones((16,), jnp.float32)

    def zh(i, _):
        hist[pl.ds(i * 16, 16)] = zeros
        return 0
    lax.fori_loop(0, NPAD // 16, zh, 0)

    def count(i, _):
        idx = cbuf[i // GP, pl.ds((i % GP) * 16, 16)]
        plsc.addupdate_scatter(hist, [idx], ones)
        return 0
    lax.fori_loop(0, CH * GP, count, 0)

    pltpu.sync_copy(hist, sh.at[s])
    plsc.subcore_barrier()

    base = s * RPS

    def za(j, _):
        acc[pl.ds(j * 16, 16)] = zeros
        return 0
    lax.fori_loop(0, RPS // 16, za, 0)

    def combine(k, _):
        pltpu.sync_copy(sh.at[k, pl.ds(base, RPS)], tbuf)

        def addj(j, _):
            acc[pl.ds(j * 16, 16)] += tbuf[pl.ds(j * 16, 16)]
            return 0
        lax.fori_loop(0, RPS // 16, addj, 0)
        return 0
    lax.fori_loop(0, NS, combine, 0)

    pltpu.sync_copy(acc, deg_out.at[c, pl.ds(base, RPS)])


@functools.partial(
    pl.kernel,
    out_type=jax.ShapeDtypeStruct((NC, NPAD, D), jnp.float32),
    mesh=_mesh,
    compiler_params=_sc_params,
    scratch_types=[
        pltpu.VMEM((2, IB, EC), jnp.int32),
        pltpu.VMEM((EC, D), jnp.float32),
        pltpu.VMEM_SHARED((NPAD, D), jnp.float32),
        pltpu.SemaphoreType.DMA,
    ],
)
def _scatter_kernel(z_hbm, row_hbm, col_hbm, s_out, rc, gbuf, acc, sem):
    c = lax.axis_index("c")
    s = lax.axis_index("s")

    zeros = jnp.zeros((16,), jnp.float32)

    def zg(i, _):
        gbuf[i // 8, pl.ds((i % 8) * 16, 16)] = zeros
        return 0
    lax.fori_loop(0, EC * 8, zg, 0)

    base = s * RPS

    def zacc(i, _):
        pltpu.sync_copy(gbuf.at[pl.ds(0, WB)],
                        acc.at[pl.ds(base + i * WB, WB)])
        return 0
    lax.fori_loop(0, RPS // WB, zacc, 0)
    plsc.subcore_barrier()

    roff = jnp.where(c == 0, s * C0, NS * C0 + s * C1)
    nblk = jnp.where(c == 0, C0 // IB, C1 // IB)

    def block(b, _):
        pltpu.sync_copy(row_hbm.at[pl.ds(roff + b * IB, IB)], rc.at[0])
        pltpu.sync_copy(col_hbm.at[pl.ds(roff + b * IB, IB)], rc.at[1])

        def chunk(j, _):
            pltpu.async_copy(z_hbm.at[rc.at[0, j]], gbuf, sem).wait()
            pltpu.sync_copy(gbuf, acc.at[rc.at[1, j]], add=True)
            return 0
        lax.fori_loop(0, IB, chunk, 0)
        return 0
    lax.fori_loop(0, nblk, block, 0)

    plsc.subcore_barrier()

    def wb(i, _):
        r0 = base + i * WB
        pltpu.sync_copy(acc.at[pl.ds(r0, WB)], gbuf.at[pl.ds(0, WB)])
        pltpu.sync_copy(gbuf.at[pl.ds(0, WB)], s_out.at[c, pl.ds(r0, WB)])
        return 0
    lax.fori_loop(0, RPS // WB, wb, 0)


@functools.partial(
    pl.kernel,
    out_type=jax.ShapeDtypeStruct((NW, CH, EC), jnp.float32),
    mesh=_mesh,
    compiler_params=_sc_params,
    scratch_types=[
        pltpu.VMEM((CH, EC), jnp.int32),
        pltpu.VMEM((CH, EC), jnp.int32),
        pltpu.VMEM((NPAD,), jnp.float32),
        pltpu.VMEM((NPAD,), jnp.float32),
        pltpu.VMEM((CH, EC), jnp.float32),
    ],
)
def _edge_kernel(ab_hbm, row_hbm, col_hbm, out, rbuf, cbuf, abuf, bbuf, obuf):
    c = lax.axis_index("c")
    s = lax.axis_index("s")
    wid = c * NS + s
    pltpu.sync_copy(row_hbm.at[wid], rbuf)
    pltpu.sync_copy(col_hbm.at[wid], cbuf)
    pltpu.sync_copy(ab_hbm.at[0], abuf)
    pltpu.sync_copy(ab_hbm.at[1], bbuf)

    def body(i, _):
        ch = i // GP
        j = (i % GP) * 16
        ri = rbuf[ch, pl.ds(j, 16)]
        ci = cbuf[ch, pl.ds(j, 16)]
        t = plsc.load_gather(abuf, [ri]) + plsc.load_gather(bbuf, [ci])
        obuf[ch, pl.ds(j, 16)] = 1.0 / (1.0 + jnp.exp(-t))
        return 0
    lax.fori_loop(0, CH * GP, body, 0)

    pltpu.sync_copy(obuf, out.at[wid])


R = 1000
_PREC = lax.Precision.HIGHEST


def _dis(deg_ref):
    return lax.rsqrt(deg_ref[:, 0] + deg_ref[:, 1] + 1.0)


def _tc1_body(deg_ref, x_ref, fcw_ref, fcb_ref, w1_ref, z1_ref):
    dis = _dis(deg_ref)
    h0 = jnp.dot(x_ref[...], fcw_ref[...], precision=_PREC,
                 preferred_element_type=jnp.float32) + fcb_ref[...]
    y1 = jnp.dot(h0, w1_ref[...], precision=_PREC,
                 preferred_element_type=jnp.float32)
    z1_ref[...] = dis[:, None] * y1


def _tc2_body(deg_ref, s_ref, z1_ref, b1_ref, w2_ref, z2_ref):
    dis = _dis(deg_ref)
    m = s_ref[0] + s_ref[1] + z1_ref[...]
    h1 = jnp.maximum(dis[:, None] * m + b1_ref[...], 0.0)
    y2 = jnp.dot(h1, w2_ref[...], precision=_PREC,
                 preferred_element_type=jnp.float32)
    z2_ref[...] = dis[:, None] * y2


def _tc3_body(deg_ref, s_ref, z2_ref, b2_ref, nsw_ref, nsb_ref, esw_ref,
              esb_ref, ns_out, ab_out):
    dis = _dis(deg_ref)
    m = s_ref[0] + s_ref[1] + z2_ref[...]
    h2 = jnp.maximum(dis[:, None] * m + b2_ref[...], 0.0)
    nl = jnp.dot(h2, nsw_ref[...], precision=_PREC,
                 preferred_element_type=jnp.float32) + nsb_ref[...]
    ns_out[...] = 1.0 / (1.0 + jnp.exp(-nl))
    ab = jnp.dot(h2, esw_ref[...], precision=_PREC,
                 preferred_element_type=jnp.float32)
    ab_out[...] = ab + esb_ref[...]


_deg_spec = pl.BlockSpec((R, 2), lambda i: (i, 0))
_row_spec = pl.BlockSpec((R, D), lambda i: (i, 0))
_w_spec = pl.BlockSpec((D, D), lambda i: (0, 0))
_b_spec = pl.BlockSpec((1, D), lambda i: (0, 0))
_s_spec = pl.BlockSpec((2, R, D), lambda i: (0, i, 0))

_tc1 = pl.pallas_call(
    _tc1_body,
    grid=(N // R,),
    in_specs=[_deg_spec, _row_spec, _w_spec, _b_spec, _w_spec],
    out_specs=_row_spec,
    out_shape=jax.ShapeDtypeStruct((N, D), jnp.float32),
)

_tc2 = pl.pallas_call(
    _tc2_body,
    grid=(N // R,),
    in_specs=[_deg_spec, _s_spec, _row_spec, _b_spec, _w_spec],
    out_specs=_row_spec,
    out_shape=jax.ShapeDtypeStruct((N, D), jnp.float32),
)

_tc3 = pl.pallas_call(
    _tc3_body,
    grid=(N // R,),
    in_specs=[
        _deg_spec, _s_spec, _row_spec, _b_spec,
        pl.BlockSpec((D, 1), lambda i: (0, 0)),
        pl.BlockSpec((1, 1), lambda i: (0, 0)),
        pl.BlockSpec((D, 2), lambda i: (0, 0)),
        pl.BlockSpec((1, 2), lambda i: (0, 0)),
    ],
    out_specs=[
        pl.BlockSpec((R, 1), lambda i: (i, 0)),
        pl.BlockSpec((R, 2), lambda i: (i, 0)),
    ],
    out_shape=[
        jax.ShapeDtypeStruct((N, 1), jnp.float32),
        jax.ShapeDtypeStruct((N, 2), jnp.float32),
    ],
)


def kernel(x, edge_index, fc_W, fc_b, w1, b1, w2, b2, ns_W, ns_b, es_W, es_b):
    row = edge_index[0].astype(jnp.int32)
    col = edge_index[1].astype(jnp.int32)
    pad = E_PAD - E
    rowp = jnp.concatenate([row, jnp.zeros((pad,), jnp.int32)])
    colp = jnp.concatenate([col, jnp.full((pad,), N, jnp.int32)])
    row3 = rowp.reshape(NW, CH, EC)
    col3 = colp.reshape(NW, CH, EC)
    row2 = rowp.reshape(TCH, EC)
    col2 = colp.reshape(TCH, EC)

    deg = _deg_kernel(col3)[:, :N].T

    z1 = _tc1(deg, x, fc_W, fc_b.reshape(1, D), w1)
    s1 = _scatter_kernel(z1, row2, col2)[:, :N]
    z2 = _tc2(deg, s1, z1, b1.reshape(1, D), w2)
    s2 = _scatter_kernel(z2, row2, col2)[:, :N]

    esw2 = jnp.concatenate([es_W[:D], es_W[D:]], axis=1)
    esb2 = jnp.stack([es_b, jnp.zeros_like(es_b)]).reshape(1, 2)
    node_score, ab = _tc3(deg, s2, z2, b2.reshape(1, D), ns_W,
                          ns_b.reshape(1, 1), esw2, esb2)

    abp = jnp.pad(ab.T, ((0, 0), (0, NPAD - N)))
    es = _edge_kernel(abp, row3, col3)
    edge_score = es.reshape(-1)[:E].reshape(E, 1)
    return (edge_score, node_score)


# speedup vs baseline: 1.3170x; 1.3170x over previous
import functools

import jax
import jax.numpy as jnp
from jax import lax
from jax.experimental import pallas as pl
from jax.experimental.pallas import tpu as pltpu
from jax.experimental.pallas import tpu_sc as plsc

N = 10000
D = 128
E = 320000

NC = 2
NS = 16
NW = NC * NS

EC = 128
CH = 80
IB = 8
C0 = 120
C1 = 40
TCH = NS * (C0 + C1)
GP = EC // 16
WB = 128
EPT = CH * EC
E_PAD = NW * EPT
NPAD = 10240
RPS = NPAD // NS

_mesh = plsc.VectorSubcoreMesh(core_axis_name="c", subcore_axis_name="s")
_sc_params = pltpu.CompilerParams(needs_layout_passes=False,
                                  use_tc_tiling_on_sc=False)


@functools.partial(
    pl.kernel,
    out_type=jax.ShapeDtypeStruct((NC, NPAD), jnp.float32),
    mesh=_mesh,
    compiler_params=_sc_params,
    scratch_types=[
        pltpu.VMEM((CH, EC), jnp.int32),
        pltpu.VMEM((NPAD,), jnp.float32),
        pltpu.VMEM((RPS,), jnp.float32),
        pltpu.VMEM((RPS,), jnp.float32),
        pltpu.VMEM_SHARED((NS, NPAD), jnp.float32),
    ],
)
def _deg_kernel(col_hbm, deg_out, cbuf, hist, tbuf, acc, sh):
    c = lax.axis_index("c")
    s = lax.axis_index("s")
    wid = c * NS + s
    pltpu.sync_copy(col_hbm.at[wid], cbuf)
    zeros = jnp.zeros((16,), jnp.float32)
    ones = jnp.ones((16,), jnp.float32)

    def zh(i, _):
        hist[pl.ds(i * 16, 16)] = zeros
        return 0
    lax.fori_loop(0, NPAD // 16, zh, 0)

    def count(i, _):
        idx = cbuf[i // GP, pl.ds((i % GP) * 16, 16)]
        plsc.addupdate_scatter(hist, [idx], ones)
        return 0
    lax.fori_loop(0, CH * GP, count, 0)

    pltpu.sync_copy(hist, sh.at[s])
    plsc.subcore_barrier()

    base = s * RPS

    def za(j, _):
        acc[pl.ds(j * 16, 16)] = zeros
        return 0
    lax.fori_loop(0, RPS // 16, za, 0)

    def combine(k, _):
        pltpu.sync_copy(sh.at[k, pl.ds(base, RPS)], tbuf)

        def addj(j, _):
            acc[pl.ds(j * 16, 16)] += tbuf[pl.ds(j * 16, 16)]
            return 0
        lax.fori_loop(0, RPS // 16, addj, 0)
        return 0
    lax.fori_loop(0, NS, combine, 0)

    pltpu.sync_copy(acc, deg_out.at[c, pl.ds(base, RPS)])


@functools.partial(
    pl.kernel,
    out_type=jax.ShapeDtypeStruct((NC, NPAD, D), jnp.float32),
    mesh=_mesh,
    compiler_params=_sc_params,
    scratch_types=[
        pltpu.VMEM((2, IB, EC), jnp.int32),
        pltpu.VMEM((EC, D), jnp.float32),
        pltpu.VMEM_SHARED((NPAD, D), jnp.float32),
        pltpu.SemaphoreType.DMA,
    ],
)
def _scatter_kernel(z_hbm, row_hbm, col_hbm, s_out, rc, gbuf, acc, sem):
    c = lax.axis_index("c")
    s = lax.axis_index("s")

    zeros = jnp.zeros((16,), jnp.float32)

    def zg(i, _):
        gbuf[i // 8, pl.ds((i % 8) * 16, 16)] = zeros
        return 0
    lax.fori_loop(0, EC * 8, zg, 0)

    base = s * RPS

    def zacc(i, _):
        pltpu.sync_copy(gbuf.at[pl.ds(0, WB)],
                        acc.at[pl.ds(base + i * WB, WB)])
        return 0
    lax.fori_loop(0, RPS // WB, zacc, 0)
    plsc.subcore_barrier()

    roff = jnp.where(c == 0, s * C0, NS * C0 + s * C1)
    nblk = jnp.where(c == 0, C0 // IB, C1 // IB)

    def block(b, _):
        pltpu.sync_copy(row_hbm.at[pl.ds(roff + b * IB, IB)], rc.at[0])
        pltpu.sync_copy(col_hbm.at[pl.ds(roff + b * IB, IB)], rc.at[1])

        def chunk(j, _):
            pltpu.async_copy(z_hbm.at[rc.at[0, j]], gbuf, sem).wait()
            pltpu.sync_copy(gbuf, acc.at[rc.at[1, j]], add=True)
            return 0
        lax.fori_loop(0, IB, chunk, 0)
        return 0
    lax.fori_loop(0, nblk, block, 0)

    plsc.subcore_barrier()

    def wb(i, _):
        r0 = base + i * WB
        pltpu.sync_copy(acc.at[pl.ds(r0, WB)], gbuf.at[pl.ds(0, WB)])
        pltpu.sync_copy(gbuf.at[pl.ds(0, WB)], s_out.at[c, pl.ds(r0, WB)])
        return 0
    lax.fori_loop(0, RPS // WB, wb, 0)


@functools.partial(
    pl.kernel,
    out_type=jax.ShapeDtypeStruct((NW, CH, EC), jnp.float32),
    mesh=_mesh,
    compiler_params=_sc_params,
    scratch_types=[
        pltpu.VMEM((CH, EC), jnp.int32),
        pltpu.VMEM((CH, EC), jnp.int32),
        pltpu.VMEM((NPAD,), jnp.float32),
        pltpu.VMEM((NPAD,), jnp.float32),
        pltpu.VMEM((CH, EC), jnp.float32),
    ],
)
def _edge_kernel(ab_hbm, row_hbm, col_hbm, out, rbuf, cbuf, abuf, bbuf, obuf):
    c = lax.axis_index("c")
    s = lax.axis_index("s")
    wid = c * NS + s
    pltpu.sync_copy(row_hbm.at[wid], rbuf)
    pltpu.sync_copy(col_hbm.at[wid], cbuf)
    pltpu.sync_copy(ab_hbm.at[0], abuf)
    pltpu.sync_copy(ab_hbm.at[1], bbuf)

    def body(i, _):
        ch = i // GP
        j = (i % GP) * 16
        ri = rbuf[ch, pl.ds(j, 16)]
        ci = cbuf[ch, pl.ds(j, 16)]
        t = plsc.load_gather(abuf, [ri]) + plsc.load_gather(bbuf, [ci])
        obuf[ch, pl.ds(j, 16)] = 1.0 / (1.0 + jnp.exp(-t))
        return 0
    lax.fori_loop(0, CH * GP, body, 0)

    pltpu.sync_copy(obuf, out.at[wid])


R = 1000
_PREC = lax.Precision.HIGHEST


def _dis(deg_ref):
    return lax.rsqrt(deg_ref[:, 0] + deg_ref[:, 1] + 1.0)


def _tc1_body(deg_ref, x_ref, fcw_ref, fcb_ref, w1_ref, z1_ref):
    dis = _dis(deg_ref)
    h0 = jnp.dot(x_ref[...], fcw_ref[...], precision=_PREC,
                 preferred_element_type=jnp.float32) + fcb_ref[...]
    y1 = jnp.dot(h0, w1_ref[...], precision=_PREC,
                 preferred_element_type=jnp.float32)
    z1_ref[...] = dis[:, None] * y1


def _tc2_body(deg_ref, s_ref, z1_ref, b1_ref, w2_ref, z2_ref):
    dis = _dis(deg_ref)
    m = s_ref[0] + s_ref[1] + z1_ref[...]
    h1 = jnp.maximum(dis[:, None] * m + b1_ref[...], 0.0)
    y2 = jnp.dot(h1, w2_ref[...], precision=_PREC,
                 preferred_element_type=jnp.float32)
    z2_ref[...] = dis[:, None] * y2


def _tc3_body(deg_ref, s_ref, z2_ref, b2_ref, nsw_ref, nsb_ref, esw_ref,
              esb_ref, ns_out, ab_out):
    dis = _dis(deg_ref)
    m = s_ref[0] + s_ref[1] + z2_ref[...]
    h2 = jnp.maximum(dis[:, None] * m + b2_ref[...], 0.0)
    nl = jnp.dot(h2, nsw_ref[...], precision=_PREC,
                 preferred_element_type=jnp.float32) + nsb_ref[...]
    ns_out[...] = 1.0 / (1.0 + jnp.exp(-nl))
    ab = jnp.dot(h2, esw_ref[...], precision=_PREC,
                 preferred_element_type=jnp.float32)
    ab_out[...] = ab + esb_ref[...]


_deg_spec = pl.BlockSpec((R, 2), lambda i: (i, 0))
_row_spec = pl.BlockSpec((R, D), lambda i: (i, 0))
_w_spec = pl.BlockSpec((D, D), lambda i: (0, 0))
_b_spec = pl.BlockSpec((1, D), lambda i: (0, 0))
_s_spec = pl.BlockSpec((2, R, D), lambda i: (0, i, 0))

_tc1 = pl.pallas_call(
    _tc1_body,
    grid=(N // R,),
    in_specs=[_deg_spec, _row_spec, _w_spec, _b_spec, _w_spec],
    out_specs=_row_spec,
    out_shape=jax.ShapeDtypeStruct((N, D), jnp.float32),
)

_tc2 = pl.pallas_call(
    _tc2_body,
    grid=(N // R,),
    in_specs=[_deg_spec, _s_spec, _row_spec, _b_spec, _w_spec],
    out_specs=_row_spec,
    out_shape=jax.ShapeDtypeStruct((N, D), jnp.float32),
)

_tc3 = pl.pallas_call(
    _tc3_body,
    grid=(N // R,),
    in_specs=[
        _deg_spec, _s_spec, _row_spec, _b_spec,
        pl.BlockSpec((D, 1), lambda i: (0, 0)),
        pl.BlockSpec((1, 1), lambda i: (0, 0)),
        pl.BlockSpec((D, 2), lambda i: (0, 0)),
        pl.BlockSpec((1, 2), lambda i: (0, 0)),
    ],
    out_specs=[
        pl.BlockSpec((R, 1), lambda i: (i, 0)),
        pl.BlockSpec((R, 2), lambda i: (i, 0)),
    ],
    out_shape=[
        jax.ShapeDtypeStruct((N, 1), jnp.float32),
        jax.ShapeDtypeStruct((N, 2), jnp.float32),
    ],
)


def kernel(x, edge_index, fc_W, fc_b, w1, b1, w2, b2, ns_W, ns_b, es_W, es_b):
    row = edge_index[0].astype(jnp.int32)
    col = edge_index[1].astype(jnp.int32)
    pad = E_PAD - E
    rowp = jnp.concatenate([row, jnp.zeros((pad,), jnp.int32)])
    colp = jnp.concatenate([col, jnp.full((pad,), N, jnp.int32)])
    row3 = rowp.reshape(NW, CH, EC)
    col3 = colp.reshape(NW, CH, EC)
    row2 = rowp.reshape(TCH, EC)
    col2 = colp.reshape(TCH, EC)

    deg = _deg_kernel(col3)[:, :N].T

    z1 = _tc1(deg, x, fc_W, fc_b.reshape(1, D), w1)
    s1 = _scatter_kernel(z1, row2, col2)[:, :N]
    z2 = _tc2(deg, s1, z1, b1.reshape(1, D), w2)
    s2 = _scatter_kernel(z2, row2, col2)[:, :N]

    esw2 = jnp.concatenate([es_W[:D], es_W[D:]], axis=1)
    esb2 = jnp.stack([es_b, jnp.zeros_like(es_b)]).reshape(1, 2)
    node_score, ab = _tc3(deg, s2, z2, b2.reshape(1, D), ns_W,
                          ns_b.reshape(1, 1), esw2, esb2)

    abp = jnp.pad(ab.T, ((0, 0), (0, NPAD - N)))
    es = _edge_kernel(abp, row3, col3)
    edge_score = es.reshape(-1)[:E].reshape(E, 1)
    return (edge_score, node_score)


# speedup vs baseline: 1.3604x; 1.0329x over previous
import functools

import jax
import jax.numpy as jnp
from jax import lax
from jax.experimental import pallas as pl
from jax.experimental.pallas import tpu as pltpu
from jax.experimental.pallas import tpu_sc as plsc

N = 10000
D = 128
E = 320000

NC = 2
NS = 16
NW = NC * NS

EC = 128
CH = 80
IB = 8
C0 = 120
C1 = 40
TCH = NS * (C0 + C1)
GP = EC // 16
WB = 128
EPT = CH * EC
E_PAD = NW * EPT
NPAD = 10240
RPS = NPAD // NS

_mesh = plsc.VectorSubcoreMesh(core_axis_name="c", subcore_axis_name="s")
_sc_params = pltpu.CompilerParams(needs_layout_passes=False,
                                  use_tc_tiling_on_sc=False)


@functools.partial(
    pl.kernel,
    out_type=jax.ShapeDtypeStruct((NC, NPAD), jnp.float32),
    mesh=_mesh,
    compiler_params=_sc_params,
    scratch_types=[
        pltpu.VMEM((CH, EC), jnp.int32),
        pltpu.VMEM((NPAD,), jnp.float32),
        pltpu.VMEM((RPS,), jnp.float32),
        pltpu.VMEM((RPS,), jnp.float32),
        pltpu.VMEM_SHARED((NS, NPAD), jnp.float32),
    ],
)
def _deg_kernel(col_hbm, deg_out, cbuf, hist, tbuf, acc, sh):
    c = lax.axis_index("c")
    s = lax.axis_index("s")
    wid = c * NS + s
    pltpu.sync_copy(col_hbm.at[wid], cbuf)
    zeros = jnp.zeros((16,), jnp.float32)
    ones = jnp.ones((16,), jnp.float32)

    def zh(i, _):
        hist[pl.ds(i * 16, 16)] = zeros
        return 0
    lax.fori_loop(0, NPAD // 16, zh, 0)

    def count(i, _):
        idx = cbuf[i // GP, pl.ds((i % GP) * 16, 16)]
        plsc.addupdate_scatter(hist, [idx], ones)
        return 0
    lax.fori_loop(0, CH * GP, count, 0)

    pltpu.sync_copy(hist, sh.at[s])
    plsc.subcore_barrier()

    base = s * RPS

    def za(j, _):
        acc[pl.ds(j * 16, 16)] = zeros
        return 0
    lax.fori_loop(0, RPS // 16, za, 0)

    def combine(k, _):
        pltpu.sync_copy(sh.at[k, pl.ds(base, RPS)], tbuf)

        def addj(j, _):
            acc[pl.ds(j * 16, 16)] += tbuf[pl.ds(j * 16, 16)]
            return 0
        lax.fori_loop(0, RPS // 16, addj, 0)
        return 0
    lax.fori_loop(0, NS, combine, 0)

    pltpu.sync_copy(acc, deg_out.at[c, pl.ds(base, RPS)])


@functools.partial(
    pl.kernel,
    out_type=jax.ShapeDtypeStruct((NC, NPAD, D), jnp.float32),
    mesh=_mesh,
    compiler_params=_sc_params,
    scratch_types=[
        pltpu.VMEM((C0, EC), jnp.int32),
        pltpu.VMEM((2, EC), jnp.int32),
        pltpu.VMEM((EC, D), jnp.float32),
        pltpu.VMEM_SHARED((NPAD, D), jnp.float32),
        pltpu.SemaphoreType.DMA,
    ],
)
def _scatter_kernel(z_hbm, pk_hbm, s_out, pk, ub, gbuf, acc, sem):
    c = lax.axis_index("c")
    s = lax.axis_index("s")

    zeros = jnp.zeros((16,), jnp.float32)

    def zg(i, _):
        gbuf[i // 8, pl.ds((i % 8) * 16, 16)] = zeros
        return 0
    lax.fori_loop(0, EC * 8, zg, 0)

    base = s * RPS

    def zacc(i, _):
        pltpu.sync_copy(gbuf.at[pl.ds(0, WB)],
                        acc.at[pl.ds(base + i * WB, WB)])
        return 0
    lax.fori_loop(0, RPS // WB, zacc, 0)
    plsc.subcore_barrier()

    @pl.when(c == 0)
    def _():
        pltpu.sync_copy(pk_hbm.at[pl.ds(s * C0, C0)], pk.at[pl.ds(0, C0)])

    @pl.when(c == 1)
    def _():
        pltpu.sync_copy(pk_hbm.at[pl.ds(NS * C0 + s * C1, C1)],
                        pk.at[pl.ds(0, C1)])

    nch = jnp.where(c == 0, C0, C1)

    def chunk(j, _):
        for g in range(GP):
            v = pk[j, pl.ds(g * 16, 16)]
            ub[0, pl.ds(g * 16, 16)] = v & 0xFFFF
            ub[1, pl.ds(g * 16, 16)] = lax.shift_right_logical(v, 16)
        pltpu.async_copy(z_hbm.at[ub.at[0]], gbuf, sem).wait()
        pltpu.sync_copy(gbuf, acc.at[ub.at[1]], add=True)
        return 0
    lax.fori_loop(0, nch, chunk, 0)

    plsc.subcore_barrier()

    def wb(i, _):
        r0 = base + i * WB
        pltpu.sync_copy(acc.at[pl.ds(r0, WB)], gbuf.at[pl.ds(0, WB)])
        pltpu.sync_copy(gbuf.at[pl.ds(0, WB)], s_out.at[c, pl.ds(r0, WB)])
        return 0
    lax.fori_loop(0, RPS // WB, wb, 0)


@functools.partial(
    pl.kernel,
    out_type=jax.ShapeDtypeStruct((NW, CH, EC), jnp.float32),
    mesh=_mesh,
    compiler_params=_sc_params,
    scratch_types=[
        pltpu.VMEM((CH, EC), jnp.int32),
        pltpu.VMEM((CH, EC), jnp.int32),
        pltpu.VMEM((NPAD,), jnp.float32),
        pltpu.VMEM((NPAD,), jnp.float32),
        pltpu.VMEM((CH, EC), jnp.float32),
    ],
)
def _edge_kernel(ab_hbm, row_hbm, col_hbm, out, rbuf, cbuf, abuf, bbuf, obuf):
    c = lax.axis_index("c")
    s = lax.axis_index("s")
    wid = c * NS + s
    pltpu.sync_copy(row_hbm.at[wid], rbuf)
    pltpu.sync_copy(col_hbm.at[wid], cbuf)
    pltpu.sync_copy(ab_hbm.at[0], abuf)
    pltpu.sync_copy(ab_hbm.at[1], bbuf)

    def body(i, _):
        ch = i // GP
        j = (i % GP) * 16
        ri = rbuf[ch, pl.ds(j, 16)]
        ci = cbuf[ch, pl.ds(j, 16)]
        t = plsc.load_gather(abuf, [ri]) + plsc.load_gather(bbuf, [ci])
        obuf[ch, pl.ds(j, 16)] = 1.0 / (1.0 + jnp.exp(-t))
        return 0
    lax.fori_loop(0, CH * GP, body, 0)

    pltpu.sync_copy(obuf, out.at[wid])


R = 1000
_PREC = lax.Precision.HIGHEST


def _dis(deg_ref):
    return lax.rsqrt(deg_ref[:, 0] + deg_ref[:, 1] + 1.0)


def _tc1_body(deg_ref, x_ref, fcw_ref, fcb_ref, w1_ref, z1_ref):
    dis = _dis(deg_ref)
    h0 = jnp.dot(x_ref[...], fcw_ref[...], precision=_PREC,
                 preferred_element_type=jnp.float32) + fcb_ref[...]
    y1 = jnp.dot(h0, w1_ref[...], precision=_PREC,
                 preferred_element_type=jnp.float32)
    z1_ref[...] = dis[:, None] * y1


def _tc2_body(deg_ref, s_ref, z1_ref, b1_ref, w2_ref, z2_ref):
    dis = _dis(deg_ref)
    m = s_ref[0] + s_ref[1] + z1_ref[...]
    h1 = jnp.maximum(dis[:, None] * m + b1_ref[...], 0.0)
    y2 = jnp.dot(h1, w2_ref[...], precision=_PREC,
                 preferred_element_type=jnp.float32)
    z2_ref[...] = dis[:, None] * y2


def _tc3_body(deg_ref, s_ref, z2_ref, b2_ref, nsw_ref, nsb_ref, esw_ref,
              esb_ref, ns_out, ab_out):
    dis = _dis(deg_ref)
    m = s_ref[0] + s_ref[1] + z2_ref[...]
    h2 = jnp.maximum(dis[:, None] * m + b2_ref[...], 0.0)
    nl = jnp.dot(h2, nsw_ref[...], precision=_PREC,
                 preferred_element_type=jnp.float32) + nsb_ref[...]
    ns_out[...] = 1.0 / (1.0 + jnp.exp(-nl))
    ab = jnp.dot(h2, esw_ref[...], precision=_PREC,
                 preferred_element_type=jnp.float32)
    ab_out[...] = ab + esb_ref[...]


_deg_spec = pl.BlockSpec((R, 2), lambda i: (i, 0))
_row_spec = pl.BlockSpec((R, D), lambda i: (i, 0))
_w_spec = pl.BlockSpec((D, D), lambda i: (0, 0))
_b_spec = pl.BlockSpec((1, D), lambda i: (0, 0))
_s_spec = pl.BlockSpec((2, R, D), lambda i: (0, i, 0))

_tc1 = pl.pallas_call(
    _tc1_body,
    grid=(N // R,),
    in_specs=[_deg_spec, _row_spec, _w_spec, _b_spec, _w_spec],
    out_specs=_row_spec,
    out_shape=jax.ShapeDtypeStruct((N, D), jnp.float32),
)

_tc2 = pl.pallas_call(
    _tc2_body,
    grid=(N // R,),
    in_specs=[_deg_spec, _s_spec, _row_spec, _b_spec, _w_spec],
    out_specs=_row_spec,
    out_shape=jax.ShapeDtypeStruct((N, D), jnp.float32),
)

_tc3 = pl.pallas_call(
    _tc3_body,
    grid=(N // R,),
    in_specs=[
        _deg_spec, _s_spec, _row_spec, _b_spec,
        pl.BlockSpec((D, 1), lambda i: (0, 0)),
        pl.BlockSpec((1, 1), lambda i: (0, 0)),
        pl.BlockSpec((D, 2), lambda i: (0, 0)),
        pl.BlockSpec((1, 2), lambda i: (0, 0)),
    ],
    out_specs=[
        pl.BlockSpec((R, 1), lambda i: (i, 0)),
        pl.BlockSpec((R, 2), lambda i: (i, 0)),
    ],
    out_shape=[
        jax.ShapeDtypeStruct((N, 1), jnp.float32),
        jax.ShapeDtypeStruct((N, 2), jnp.float32),
    ],
)


def kernel(x, edge_index, fc_W, fc_b, w1, b1, w2, b2, ns_W, ns_b, es_W, es_b):
    row = edge_index[0].astype(jnp.int32)
    col = edge_index[1].astype(jnp.int32)
    pad = E_PAD - E
    rowp = jnp.concatenate([row, jnp.zeros((pad,), jnp.int32)])
    colp = jnp.concatenate([col, jnp.full((pad,), N, jnp.int32)])
    row3 = rowp.reshape(NW, CH, EC)
    col3 = colp.reshape(NW, CH, EC)
    pk2 = (rowp | (colp << 16)).reshape(TCH, EC)

    deg = _deg_kernel(col3)[:, :N].T

    z1 = _tc1(deg, x, fc_W, fc_b.reshape(1, D), w1)
    s1 = _scatter_kernel(z1, pk2)[:, :N]
    z2 = _tc2(deg, s1, z1, b1.reshape(1, D), w2)
    s2 = _scatter_kernel(z2, pk2)[:, :N]

    esw2 = jnp.concatenate([es_W[:D], es_W[D:]], axis=1)
    esb2 = jnp.stack([es_b, jnp.zeros_like(es_b)]).reshape(1, 2)
    node_score, ab = _tc3(deg, s2, z2, b2.reshape(1, D), ns_W,
                          ns_b.reshape(1, 1), esw2, esb2)

    abp = jnp.pad(ab.T, ((0, 0), (0, NPAD - N)))
    es = _edge_kernel(abp, row3, col3)
    edge_score = es.reshape(-1)[:E].reshape(E, 1)
    return (edge_score, node_score)


# speedup vs baseline: 1.6370x; 1.2033x over previous
import functools

import jax
import jax.numpy as jnp
from jax import lax
from jax.experimental import pallas as pl
from jax.experimental.pallas import tpu as pltpu
from jax.experimental.pallas import tpu_sc as plsc

N = 10000
D = 128
E = 320000

NC = 2
NS = 16
NW = NC * NS

EC = 128
CH = 79
GP = EC // 16
WB = 128
EPT = CH * EC
E_PAD = NW * EPT
NPAD = 10240
RPS = NPAD // NS

_mesh = plsc.VectorSubcoreMesh(core_axis_name="c", subcore_axis_name="s")
_sc_params = pltpu.CompilerParams(needs_layout_passes=False)


@functools.partial(
    pl.kernel,
    out_type=jax.ShapeDtypeStruct((NC, NPAD), jnp.float32),
    mesh=_mesh,
    compiler_params=_sc_params,
    scratch_types=[
        pltpu.VMEM((CH, EC), jnp.int32),
        pltpu.VMEM((NPAD,), jnp.float32),
        pltpu.VMEM((RPS,), jnp.float32),
        pltpu.VMEM((RPS,), jnp.float32),
        pltpu.VMEM_SHARED((NS, NPAD), jnp.float32),
    ],
)
def _deg_kernel(col_hbm, deg_out, cbuf, hist, tbuf, acc, sh):
    c = lax.axis_index("c")
    s = lax.axis_index("s")
    wid = c * NS + s
    pltpu.sync_copy(col_hbm.at[wid], cbuf)
    zeros = jnp.zeros((16,), jnp.float32)
    ones = jnp.ones((16,), jnp.float32)

    def zh(i, _):
        hist[pl.ds(i * 16, 16)] = zeros
        return 0
    lax.fori_loop(0, NPAD // 16, zh, 0)

    def count(i, _):
        idx = cbuf[i // GP, pl.ds((i % GP) * 16, 16)]
        plsc.addupdate_scatter(hist, [idx], ones)
        return 0
    lax.fori_loop(0, CH * GP, count, 0)

    pltpu.sync_copy(hist, sh.at[s])
    plsc.subcore_barrier()

    base = s * RPS

    def za(j, _):
        acc[pl.ds(j * 16, 16)] = zeros
        return 0
    lax.fori_loop(0, RPS // 16, za, 0)

    def combine(k, _):
        pltpu.sync_copy(sh.at[k, pl.ds(base, RPS)], tbuf)

        def addj(j, _):
            acc[pl.ds(j * 16, 16)] += tbuf[pl.ds(j * 16, 16)]
            return 0
        lax.fori_loop(0, RPS // 16, addj, 0)
        return 0
    lax.fori_loop(0, NS, combine, 0)

    pltpu.sync_copy(acc, deg_out.at[c, pl.ds(base, RPS)])


@functools.partial(
    pl.kernel,
    out_type=jax.ShapeDtypeStruct((NC, NPAD, D), jnp.float32),
    mesh=_mesh,
    compiler_params=_sc_params,
    scratch_types=[
        pltpu.VMEM((CH, EC), jnp.int32),
        pltpu.VMEM((CH, EC), jnp.int32),
        pltpu.VMEM((EC, D), jnp.float32),
        pltpu.VMEM_SHARED((NPAD, D), jnp.float32),
        pltpu.SemaphoreType.DMA,
    ],
)
def _scatter_kernel(z_hbm, row_hbm, col_hbm, s_out, rbuf, cbuf, gbuf, acc,
                    sem):
    c = lax.axis_index("c")
    s = lax.axis_index("s")
    wid = c * NS + s
    pltpu.sync_copy(row_hbm.at[wid], rbuf)
    pltpu.sync_copy(col_hbm.at[wid], cbuf)

    zeros = jnp.zeros((16,), jnp.float32)

    def zg(i, _):
        gbuf[i // 8, pl.ds((i % 8) * 16, 16)] = zeros
        return 0
    lax.fori_loop(0, EC * 8, zg, 0)

    base = s * RPS

    def zacc(i, _):
        pltpu.sync_copy(gbuf.at[pl.ds(0, WB)],
                        acc.at[pl.ds(base + i * WB, WB)])
        return 0
    lax.fori_loop(0, RPS // WB, zacc, 0)
    plsc.subcore_barrier()

    def chunk(j, _):
        pltpu.async_copy(z_hbm.at[rbuf.at[j]], gbuf, sem).wait()
        pltpu.sync_copy(gbuf, acc.at[cbuf.at[j]], add=True)
        return 0
    lax.fori_loop(0, CH, chunk, 0)

    plsc.subcore_barrier()

    def wb(i, _):
        r0 = base + i * WB
        pltpu.sync_copy(acc.at[pl.ds(r0, WB)], gbuf.at[pl.ds(0, WB)])
        pltpu.sync_copy(gbuf.at[pl.ds(0, WB)], s_out.at[c, pl.ds(r0, WB)])
        return 0
    lax.fori_loop(0, RPS // WB, wb, 0)


@functools.partial(
    pl.kernel,
    out_type=jax.ShapeDtypeStruct((NW, CH, EC), jnp.float32),
    mesh=_mesh,
    compiler_params=_sc_params,
    scratch_types=[
        pltpu.VMEM((CH, EC), jnp.int32),
        pltpu.VMEM((CH, EC), jnp.int32),
        pltpu.VMEM((NPAD,), jnp.float32),
        pltpu.VMEM((NPAD,), jnp.float32),
        pltpu.VMEM((CH, EC), jnp.float32),
    ],
)
def _edge_kernel(ab_hbm, row_hbm, col_hbm, out, rbuf, cbuf, abuf, bbuf, obuf):
    c = lax.axis_index("c")
    s = lax.axis_index("s")
    wid = c * NS + s
    pltpu.sync_copy(row_hbm.at[wid], rbuf)
    pltpu.sync_copy(col_hbm.at[wid], cbuf)
    pltpu.sync_copy(ab_hbm.at[0], abuf)
    pltpu.sync_copy(ab_hbm.at[1], bbuf)

    def body(i, _):
        ch = i // GP
        j = (i % GP) * 16
        ri = rbuf[ch, pl.ds(j, 16)]
        ci = cbuf[ch, pl.ds(j, 16)]
        t = plsc.load_gather(abuf, [ri]) + plsc.load_gather(bbuf, [ci])
        obuf[ch, pl.ds(j, 16)] = 1.0 / (1.0 + jnp.exp(-t))
        return 0
    lax.fori_loop(0, CH * GP, body, 0)

    pltpu.sync_copy(obuf, out.at[wid])


R = 1000
_PREC = lax.Precision.HIGHEST


def _dis(deg_ref):
    return lax.rsqrt(deg_ref[:, 0] + deg_ref[:, 1] + 1.0)


def _tc1_body(deg_ref, x_ref, fcw_ref, fcb_ref, w1_ref, z1_ref):
    dis = _dis(deg_ref)
    h0 = jnp.dot(x_ref[...], fcw_ref[...], precision=_PREC,
                 preferred_element_type=jnp.float32) + fcb_ref[...]
    y1 = jnp.dot(h0, w1_ref[...], precision=_PREC,
                 preferred_element_type=jnp.float32)
    z1_ref[...] = dis[:, None] * y1


def _tc2_body(deg_ref, s_ref, z1_ref, b1_ref, w2_ref, z2_ref):
    dis = _dis(deg_ref)
    m = s_ref[0] + s_ref[1] + z1_ref[...]
    h1 = jnp.maximum(dis[:, None] * m + b1_ref[...], 0.0)
    y2 = jnp.dot(h1, w2_ref[...], precision=_PREC,
                 preferred_element_type=jnp.float32)
    z2_ref[...] = dis[:, None] * y2


def _tc3_body(deg_ref, s_ref, z2_ref, b2_ref, nsw_ref, nsb_ref, esw_ref,
              esb_ref, ns_out, ab_out):
    dis = _dis(deg_ref)
    m = s_ref[0] + s_ref[1] + z2_ref[...]
    h2 = jnp.maximum(dis[:, None] * m + b2_ref[...], 0.0)
    nl = jnp.dot(h2, nsw_ref[...], precision=_PREC,
                 preferred_element_type=jnp.float32) + nsb_ref[...]
    ns_out[...] = 1.0 / (1.0 + jnp.exp(-nl))
    ab = jnp.dot(h2, esw_ref[...], precision=_PREC,
                 preferred_element_type=jnp.float32)
    ab_out[...] = ab + esb_ref[...]


_deg_spec = pl.BlockSpec((R, 2), lambda i: (i, 0))
_row_spec = pl.BlockSpec((R, D), lambda i: (i, 0))
_w_spec = pl.BlockSpec((D, D), lambda i: (0, 0))
_b_spec = pl.BlockSpec((1, D), lambda i: (0, 0))
_s_spec = pl.BlockSpec((2, R, D), lambda i: (0, i, 0))

_tc1 = pl.pallas_call(
    _tc1_body,
    grid=(N // R,),
    in_specs=[_deg_spec, _row_spec, _w_spec, _b_spec, _w_spec],
    out_specs=_row_spec,
    out_shape=jax.ShapeDtypeStruct((N, D), jnp.float32),
)

_tc2 = pl.pallas_call(
    _tc2_body,
    grid=(N // R,),
    in_specs=[_deg_spec, _s_spec, _row_spec, _b_spec, _w_spec],
    out_specs=_row_spec,
    out_shape=jax.ShapeDtypeStruct((N, D), jnp.float32),
)

_tc3 = pl.pallas_call(
    _tc3_body,
    grid=(N // R,),
    in_specs=[
        _deg_spec, _s_spec, _row_spec, _b_spec,
        pl.BlockSpec((D, 1), lambda i: (0, 0)),
        pl.BlockSpec((1, 1), lambda i: (0, 0)),
        pl.BlockSpec((D, 2), lambda i: (0, 0)),
        pl.BlockSpec((1, 2), lambda i: (0, 0)),
    ],
    out_specs=[
        pl.BlockSpec((R, 1), lambda i: (i, 0)),
        pl.BlockSpec((R, 2), lambda i: (i, 0)),
    ],
    out_shape=[
        jax.ShapeDtypeStruct((N, 1), jnp.float32),
        jax.ShapeDtypeStruct((N, 2), jnp.float32),
    ],
)


def kernel(x, edge_index, fc_W, fc_b, w1, b1, w2, b2, ns_W, ns_b, es_W, es_b):
    row = edge_index[0].astype(jnp.int32)
    col = edge_index[1].astype(jnp.int32)
    pad = E_PAD - E
    rowp = jnp.concatenate([row, jnp.zeros((pad,), jnp.int32)])
    colp = jnp.concatenate([col, jnp.full((pad,), N, jnp.int32)])
    row3 = rowp.reshape(NW, CH, EC)
    col3 = colp.reshape(NW, CH, EC)

    deg = _deg_kernel(col3)[:, :N].T

    z1 = _tc1(deg, x, fc_W, fc_b.reshape(1, D), w1)
    s1 = _scatter_kernel(z1, row3, col3)[:, :N]
    z2 = _tc2(deg, s1, z1, b1.reshape(1, D), w2)
    s2 = _scatter_kernel(z2, row3, col3)[:, :N]

    esw2 = jnp.concatenate([es_W[:D], es_W[D:]], axis=1)
    esb2 = jnp.stack([es_b, jnp.zeros_like(es_b)]).reshape(1, 2)
    node_score, ab = _tc3(deg, s2, z2, b2.reshape(1, D), ns_W,
                          ns_b.reshape(1, 1), esw2, esb2)

    abp = jnp.pad(ab.T, ((0, 0), (0, NPAD - N)))
    es = _edge_kernel(abp, row3, col3)
    edge_score = es.reshape(-1)[:E].reshape(E, 1)
    return (edge_score, node_score)


# speedup vs baseline: 2.6073x; 1.5928x over previous
import functools

import jax
import jax.numpy as jnp
from jax import lax
from jax.experimental import pallas as pl
from jax.experimental.pallas import tpu as pltpu
from jax.experimental.pallas import tpu_sc as plsc

N = 10000
D = 128
E = 320000

NC = 2
NS = 16
NW = NC * NS

EC = 128
CH = 79
GP = EC // 16
WB = 128
EPT = CH * EC
E_PAD = NW * EPT
NPAD = 10240
RPS = NPAD // NS

_mesh = plsc.VectorSubcoreMesh(core_axis_name="c", subcore_axis_name="s")
_sc_params = pltpu.CompilerParams(needs_layout_passes=False)


@functools.partial(
    pl.kernel,
    out_type=jax.ShapeDtypeStruct((NC, NPAD), jnp.float32),
    mesh=_mesh,
    compiler_params=_sc_params,
    scratch_types=[
        pltpu.VMEM((CH, EC), jnp.int32),
        pltpu.VMEM((NPAD,), jnp.float32),
        pltpu.VMEM((RPS,), jnp.float32),
        pltpu.VMEM((RPS,), jnp.float32),
        pltpu.VMEM_SHARED((NS, NPAD), jnp.float32),
    ],
)
def _deg_kernel(col_hbm, deg_out, cbuf, hist, tbuf, acc, sh):
    c = lax.axis_index("c")
    s = lax.axis_index("s")
    wid = c * NS + s
    pltpu.sync_copy(col_hbm.at[wid], cbuf)
    zeros = jnp.zeros((16,), jnp.float32)
    ones = jnp.ones((16,), jnp.float32)

    def zh(i, _):
        hist[pl.ds(i * 16, 16)] = zeros
        return 0
    lax.fori_loop(0, NPAD // 16, zh, 0)

    def count(i, _):
        idx = cbuf[i // GP, pl.ds((i % GP) * 16, 16)]
        plsc.addupdate_scatter(hist, [idx], ones)
        return 0
    lax.fori_loop(0, CH * GP, count, 0)

    pltpu.sync_copy(hist, sh.at[s])
    plsc.subcore_barrier()

    base = s * RPS

    def za(j, _):
        acc[pl.ds(j * 16, 16)] = zeros
        return 0
    lax.fori_loop(0, RPS // 16, za, 0)

    def combine(k, _):
        pltpu.sync_copy(sh.at[k, pl.ds(base, RPS)], tbuf)

        def addj(j, _):
            acc[pl.ds(j * 16, 16)] += tbuf[pl.ds(j * 16, 16)]
            return 0
        lax.fori_loop(0, RPS // 16, addj, 0)
        return 0
    lax.fori_loop(0, NS, combine, 0)

    pltpu.sync_copy(acc, deg_out.at[c, pl.ds(base, RPS)])


@functools.partial(
    pl.kernel,
    out_type=jax.ShapeDtypeStruct((NC, NPAD, D), jnp.float32),
    mesh=_mesh,
    compiler_params=_sc_params,
    scratch_types=[
        pltpu.VMEM((CH, EC), jnp.int32),
        pltpu.VMEM((CH, EC), jnp.int32),
        pltpu.VMEM((EC, D), jnp.float32),
        pltpu.VMEM_SHARED((NPAD, D), jnp.float32),
        pltpu.SemaphoreType.DMA,
    ],
)
def _scatter_kernel(z_hbm, row_hbm, col_hbm, s_out, rbuf, cbuf, gbuf, acc,
                    sem):
    c = lax.axis_index("c")
    s = lax.axis_index("s")
    wid = c * NS + s
    pltpu.sync_copy(row_hbm.at[wid], rbuf)
    pltpu.sync_copy(col_hbm.at[wid], cbuf)

    zeros = jnp.zeros((16,), jnp.float32)

    def zg(i, _):
        gbuf[i // 8, pl.ds((i % 8) * 16, 16)] = zeros
        return 0
    lax.fori_loop(0, EC * 8, zg, 0)

    base = s * RPS

    def zacc(i, _):
        pltpu.sync_copy(gbuf.at[pl.ds(0, WB)],
                        acc.at[pl.ds(base + i * WB, WB)])
        return 0
    lax.fori_loop(0, RPS // WB, zacc, 0)
    plsc.subcore_barrier()

    def chunk(j, _):
        pltpu.async_copy(z_hbm.at[rbuf.at[j]], gbuf, sem).wait()
        pltpu.sync_copy(gbuf, acc.at[cbuf.at[j]], add=True)
        return 0
    lax.fori_loop(0, CH, chunk, 0)

    plsc.subcore_barrier()

    def wb(i, _):
        r0 = base + i * WB
        pltpu.sync_copy(acc.at[pl.ds(r0, WB)], gbuf.at[pl.ds(0, WB)])
        pltpu.sync_copy(gbuf.at[pl.ds(0, WB)], s_out.at[c, pl.ds(r0, WB)])
        return 0
    lax.fori_loop(0, RPS // WB, wb, 0)


@functools.partial(
    pl.kernel,
    out_type=jax.ShapeDtypeStruct((NW, CH, EC), jnp.float32),
    mesh=_mesh,
    compiler_params=_sc_params,
    scratch_types=[
        pltpu.VMEM((CH, EC), jnp.int32),
        pltpu.VMEM((CH, EC), jnp.int32),
        pltpu.VMEM((NPAD,), jnp.float32),
        pltpu.VMEM((NPAD,), jnp.float32),
        pltpu.VMEM((CH, EC), jnp.float32),
    ],
)
def _edge_kernel(ab_hbm, row_hbm, col_hbm, out, rbuf, cbuf, abuf, bbuf, obuf):
    c = lax.axis_index("c")
    s = lax.axis_index("s")
    wid = c * NS + s
    pltpu.sync_copy(row_hbm.at[wid], rbuf)
    pltpu.sync_copy(col_hbm.at[wid], cbuf)
    pltpu.sync_copy(ab_hbm.at[0], abuf)
    pltpu.sync_copy(ab_hbm.at[1], bbuf)

    def body(i, _):
        ch = i // GP
        j = (i % GP) * 16
        ri = rbuf[ch, pl.ds(j, 16)]
        ci = cbuf[ch, pl.ds(j, 16)]
        t = plsc.load_gather(abuf, [ri]) + plsc.load_gather(bbuf, [ci])
        obuf[ch, pl.ds(j, 16)] = 1.0 / (1.0 + jnp.exp(-t))
        return 0
    lax.fori_loop(0, CH * GP, body, 0)

    pltpu.sync_copy(obuf, out.at[wid])


R = 1000
_PREC = lax.Precision.HIGHEST


def _dis(deg_ref):
    return lax.rsqrt(deg_ref[:, 0] + deg_ref[:, 1] + 1.0)


def _tc1_body(deg_ref, x_ref, fcw_ref, fcb_ref, w1_ref, z1_ref):
    dis = _dis(deg_ref)
    h0 = jnp.dot(x_ref[...], fcw_ref[...], precision=_PREC,
                 preferred_element_type=jnp.float32) + fcb_ref[...]
    y1 = jnp.dot(h0, w1_ref[...], precision=_PREC,
                 preferred_element_type=jnp.float32)
    z1_ref[...] = dis[:, None] * y1


def _tc2_body(deg_ref, s_ref, z1_ref, b1_ref, w2_ref, z2_ref):
    dis = _dis(deg_ref)
    m = s_ref[0] + s_ref[1] + z1_ref[...]
    h1 = jnp.maximum(dis[:, None] * m + b1_ref[...], 0.0)
    y2 = jnp.dot(h1, w2_ref[...], precision=_PREC,
                 preferred_element_type=jnp.float32)
    z2_ref[...] = dis[:, None] * y2


def _tc3_body(deg_ref, s_ref, z2_ref, b2_ref, nsw_ref, nsb_ref, esw_ref,
              esb_ref, ns_out, ab_out):
    dis = _dis(deg_ref)
    m = s_ref[0] + s_ref[1] + z2_ref[...]
    h2 = jnp.maximum(dis[:, None] * m + b2_ref[...], 0.0)
    nl = jnp.dot(h2, nsw_ref[...], precision=_PREC,
                 preferred_element_type=jnp.float32) + nsb_ref[...]
    ns_out[...] = 1.0 / (1.0 + jnp.exp(-nl))
    ab = jnp.dot(h2, esw_ref[...], precision=_PREC,
                 preferred_element_type=jnp.float32)
    ab_out[...] = ab + esb_ref[...]


_deg_spec = pl.BlockSpec((R, 2), lambda i: (i, 0))
_row_spec = pl.BlockSpec((R, D), lambda i: (i, 0))
_w_spec = pl.BlockSpec((D, D), lambda i: (0, 0))
_b_spec = pl.BlockSpec((1, D), lambda i: (0, 0))
_s_spec = pl.BlockSpec((2, R, D), lambda i: (0, i, 0))

_tc1 = pl.pallas_call(
    _tc1_body,
    grid=(N // R,),
    in_specs=[_deg_spec, _row_spec, _w_spec, _b_spec, _w_spec],
    out_specs=_row_spec,
    out_shape=jax.ShapeDtypeStruct((N, D), jnp.float32),
)

_tc2 = pl.pallas_call(
    _tc2_body,
    grid=(N // R,),
    in_specs=[_deg_spec, _s_spec, _row_spec, _b_spec, _w_spec],
    out_specs=_row_spec,
    out_shape=jax.ShapeDtypeStruct((N, D), jnp.float32),
)

_tc3 = pl.pallas_call(
    _tc3_body,
    grid=(N // R,),
    in_specs=[
        _deg_spec, _s_spec, _row_spec, _b_spec,
        pl.BlockSpec((D, 1), lambda i: (0, 0)),
        pl.BlockSpec((1, 1), lambda i: (0, 0)),
        pl.BlockSpec((D, 2), lambda i: (0, 0)),
        pl.BlockSpec((1, 2), lambda i: (0, 0)),
    ],
    out_specs=[
        pl.BlockSpec((R, 1), lambda i: (i, 0)),
        pl.BlockSpec((R, 2), lambda i: (i, 0)),
    ],
    out_shape=[
        jax.ShapeDtypeStruct((N, 1), jnp.float32),
        jax.ShapeDtypeStruct((N, 2), jnp.float32),
    ],
)


def kernel(x, edge_index, fc_W, fc_b, w1, b1, w2, b2, ns_W, ns_b, es_W, es_b):
    row = edge_index[0].astype(jnp.int32)
    col = edge_index[1].astype(jnp.int32)
    pad = E_PAD - E
    pidx = jnp.arange(pad, dtype=jnp.int32)
    rowp = jnp.concatenate([row, pidx % N])
    colp = jnp.concatenate([col, N + pidx % (NPAD - N)])
    row3 = rowp.reshape(NW, CH, EC)
    col3 = colp.reshape(NW, CH, EC)

    deg = _deg_kernel(col3)[:, :N].T

    z1 = _tc1(deg, x, fc_W, fc_b.reshape(1, D), w1)
    s1 = _scatter_kernel(z1, row3, col3)[:, :N]
    z2 = _tc2(deg, s1, z1, b1.reshape(1, D), w2)
    s2 = _scatter_kernel(z2, row3, col3)[:, :N]

    esw2 = jnp.concatenate([es_W[:D], es_W[D:]], axis=1)
    esb2 = jnp.stack([es_b, jnp.zeros_like(es_b)]).reshape(1, 2)
    node_score, ab = _tc3(deg, s2, z2, b2.reshape(1, D), ns_W,
                          ns_b.reshape(1, 1), esw2, esb2)

    abp = jnp.pad(ab.T, ((0, 0), (0, NPAD - N)))
    es = _edge_kernel(abp, row3, col3)
    edge_score = es.reshape(-1)[:E].reshape(E, 1)
    return (edge_score, node_score)
